# Initial kernel scaffold; baseline (speedup 1.0000x reference)
import functools

import jax
import jax.numpy as jnp
from jax import lax
from jax.experimental import pallas as pl
from jax.experimental.pallas import tpu as pltpu
from jax.experimental.pallas import tpu_sc as plsc

NN = 10000
EE = 320000
DD = 128
GG = 128
NW = 32
EPW = EE // NW
BLK = 80
NBLK = EPW // BLK
RPT = NN // 16
NPAD = 10240
DPT = NPAD // 16

_mesh = plsc.VectorSubcoreMesh(core_axis_name="c", subcore_axis_name="s")


@functools.partial(
    pl.kernel,
    mesh=_mesh,
    out_type=jax.ShapeDtypeStruct((2, NPAD), jnp.float32),
    scratch_types=[
        pltpu.VMEM((NBLK, BLK), jnp.int32),
        pltpu.VMEM((BLK,), jnp.float32),
        pltpu.VMEM((DPT,), jnp.float32),
        pltpu.VMEM_SHARED((NPAD,), jnp.float32),
    ],
)
def _sc_degree(dst_hbm, out_hbm, dst_v, ones_v, zero_v, acc_sh):
    cid = lax.axis_index("c")
    sid = lax.axis_index("s")
    wid = cid * 16 + sid
    pltpu.sync_copy(dst_hbm.at[wid], dst_v)
    for i in range(BLK // 16):
        ones_v[pl.ds(i * 16, 16)] = jnp.full((16,), 1.0, jnp.float32)
    for i in range(DPT // 16):
        zero_v[pl.ds(i * 16, 16)] = jnp.zeros((16,), jnp.float32)
    pltpu.sync_copy(zero_v, acc_sh.at[pl.ds(sid * DPT, DPT)])
    plsc.subcore_barrier()

    def body(j, _):
        pltpu.sync_copy(ones_v, acc_sh.at[dst_v.at[j]], add=True)
        return 0

    lax.fori_loop(0, NBLK, body, 0)
    plsc.subcore_barrier()
    pltpu.sync_copy(acc_sh.at[pl.ds(sid * DPT, DPT)],
                    out_hbm.at[cid, pl.ds(sid * DPT, DPT)])


@functools.partial(
    pl.kernel,
    mesh=_mesh,
    out_type=jax.ShapeDtypeStruct((2, NN, DD), jnp.float32),
    scratch_types=[
        pltpu.VMEM((NBLK, BLK), jnp.int32),
        pltpu.VMEM((NBLK, BLK), jnp.int32),
        pltpu.VMEM((BLK, DD), jnp.float32),
        pltpu.VMEM_SHARED((NN, DD), jnp.float32),
        pltpu.SemaphoreType.DMA,
    ],
)
def _sc_edge(y_hbm, src_hbm, dst_hbm, z_hbm, out_hbm,
             src_v, dst_v, buf, acc_sh, sem):
    cid = lax.axis_index("c")
    sid = lax.axis_index("s")
    wid = cid * 16 + sid
    pltpu.sync_copy(src_hbm.at[wid], src_v)
    pltpu.sync_copy(dst_hbm.at[wid], dst_v)
    pltpu.sync_copy(z_hbm, acc_sh.at[pl.ds(sid * RPT, RPT)])
    plsc.subcore_barrier()

    def body(j, _):
        pltpu.async_copy(y_hbm.at[src_v.at[j]], buf, sem).wait()
        pltpu.sync_copy(buf, acc_sh.at[dst_v.at[j]], add=True)
        return 0

    lax.fori_loop(0, NBLK, body, 0)
    plsc.subcore_barrier()
    pltpu.sync_copy(acc_sh.at[pl.ds(sid * RPT, RPT)],
                    out_hbm.at[cid, pl.ds(sid * RPT, RPT)])


def _tc_deg_body(degp_ref, dinv_ref):
    deg = degp_ref[0] + degp_ref[1] + 1.0
    dinv_ref[...] = lax.rsqrt(deg)


def _tc_pre_body(x_ref, w_ref, dcol_ref, y_ref):
    y_ref[...] = jnp.dot(x_ref[...], w_ref[...],
                         preferred_element_type=jnp.float32) * dcol_ref[...]


def _elu(t):
    return jnp.where(t > 0, t, jnp.expm1(t))


def _bn_elu(t, g_ref, be_ref):
    m = jnp.mean(t, axis=0, keepdims=True)
    v = jnp.mean((t - m) ** 2, axis=0, keepdims=True)
    hn = (t - m) * lax.rsqrt(v + 1e-5) * g_ref[...] + be_ref[...]
    return _elu(hn)


def _tc_post_body(accp_ref, y_ref, prev_ref, dcol_ref, b_ref, g_ref, be_ref,
                  wn_ref, h_ref, ynext_ref, *, has_prev):
    dcol = dcol_ref[...]
    t = dcol * (accp_ref[0] + accp_ref[1] + y_ref[...]) + b_ref[...]
    if has_prev:
        t = t + prev_ref[...]
    h = _bn_elu(t, g_ref, be_ref)
    h_ref[...] = h
    ynext_ref[...] = jnp.dot(h, wn_ref[...],
                             preferred_element_type=jnp.float32) * dcol


def _tc_final_body(accp_ref, y_ref, prev_ref, dcol_ref, b_ref, g_ref, be_ref,
                   batch_ref, wr_ref, br_ref, out_ref):
    dcol = dcol_ref[...]
    t = dcol * (accp_ref[0] + accp_ref[1] + y_ref[...]) + b_ref[...]
    t = t + prev_ref[...]
    h = _bn_elu(t, g_ref, be_ref)
    gidx = lax.broadcasted_iota(jnp.int32, (1, GG), 1)
    mask = (batch_ref[...] == gidx).astype(jnp.float32)
    counts = jnp.sum(mask, axis=0, keepdims=True)
    maskn = mask / jnp.maximum(counts, 1.0)
    pooled = lax.dot_general(maskn, h, (((0,), (0,)), ((), ())),
                             preferred_element_type=jnp.float32)
    out_ref[...] = jnp.dot(pooled, wr_ref[...],
                           preferred_element_type=jnp.float32) + br_ref[...]


_f32 = jnp.float32
_tc_deg = pl.pallas_call(
    _tc_deg_body, out_shape=jax.ShapeDtypeStruct((NPAD // DD, DD), _f32))
_tc_pre = pl.pallas_call(
    _tc_pre_body, out_shape=jax.ShapeDtypeStruct((NN, DD), _f32))
_tc_post0 = pl.pallas_call(
    functools.partial(_tc_post_body, has_prev=False),
    out_shape=(jax.ShapeDtypeStruct((NN, DD), _f32),
               jax.ShapeDtypeStruct((NN, DD), _f32)))
_tc_post1 = pl.pallas_call(
    functools.partial(_tc_post_body, has_prev=True),
    out_shape=(jax.ShapeDtypeStruct((NN, DD), _f32),
               jax.ShapeDtypeStruct((NN, DD), _f32)))
_tc_final = pl.pallas_call(
    _tc_final_body, out_shape=jax.ShapeDtypeStruct((GG, 2), _f32))


def kernel(x, edge_index, batch, W1, b1, g1, be1, W2, b2, g2, be2,
           W3, b3, g3, be3, Wr, br):
    src2d = edge_index[0].reshape(NW, NBLK, BLK)
    dst2d = edge_index[1].reshape(NW, NBLK, BLK)
    batch_col = batch.reshape(NN, 1)
    zrows = jnp.zeros((RPT, DD), _f32)
    b1r, g1r, be1r = b1.reshape(1, DD), g1.reshape(1, DD), be1.reshape(1, DD)
    b2r, g2r, be2r = b2.reshape(1, DD), g2.reshape(1, DD), be2.reshape(1, DD)
    b3r, g3r, be3r = b3.reshape(1, DD), g3.reshape(1, DD), be3.reshape(1, DD)
    brr = br.reshape(1, 2)

    degp = _sc_degree(dst2d)
    dinv_grid = _tc_deg(degp.reshape(2, NPAD // DD, DD))
    dcol = dinv_grid.reshape(NPAD, 1)[:NN]

    y1 = _tc_pre(x, W1, dcol)
    acc1 = _sc_edge(y1, src2d, dst2d, zrows)
    h1, y2 = _tc_post0(acc1, y1, y1, dcol, b1r, g1r, be1r, W2)
    acc2 = _sc_edge(y2, src2d, dst2d, zrows)
    h2, y3 = _tc_post1(acc2, y2, x, dcol, b2r, g2r, be2r, W3)
    acc3 = _sc_edge(y3, src2d, dst2d, zrows)
    out = _tc_final(acc3, y3, h1, dcol, b3r, g3r, be3r, batch_col, Wr, brr)
    return out


# speedup vs baseline: 15.6651x; 15.6651x over previous
import functools

import jax
import jax.numpy as jnp
from jax import lax
from jax.experimental import pallas as pl
from jax.experimental.pallas import tpu as pltpu
from jax.experimental.pallas import tpu_sc as plsc

NN = 10000
EE = 320000
DD = 128
GG = 128
NW = 32
EPW = EE // NW
BLK = 80
NBLK = EPW // BLK
NPAD = 10240
RPT = NPAD // 16
DPT = NPAD // 16

_mesh = plsc.VectorSubcoreMesh(core_axis_name="c", subcore_axis_name="s")


@functools.partial(
    pl.kernel,
    mesh=_mesh,
    out_type=jax.ShapeDtypeStruct((2, 1, NPAD), jnp.float32),
    scratch_types=[
        pltpu.VMEM((NBLK, BLK), jnp.int32),
        pltpu.VMEM((BLK,), jnp.float32),
        pltpu.VMEM((DPT,), jnp.float32),
        pltpu.VMEM_SHARED((NPAD,), jnp.float32),
    ],
)
def _sc_degree(dst_hbm, out_hbm, dst_v, ones_v, zero_v, acc_sh):
    cid = lax.axis_index("c")
    sid = lax.axis_index("s")
    wid = cid * 16 + sid
    pltpu.sync_copy(dst_hbm.at[wid], dst_v)
    for i in range(BLK // 16):
        ones_v[pl.ds(i * 16, 16)] = jnp.full((16,), 1.0, jnp.float32)
    for i in range(DPT // 16):
        zero_v[pl.ds(i * 16, 16)] = jnp.zeros((16,), jnp.float32)
    pltpu.sync_copy(zero_v, acc_sh.at[pl.ds(sid * DPT, DPT)])
    plsc.subcore_barrier()

    def body(j, _):
        pltpu.sync_copy(ones_v, acc_sh.at[dst_v.at[j]], add=True)
        return 0

    lax.fori_loop(0, NBLK, body, 0)
    plsc.subcore_barrier()
    pltpu.sync_copy(acc_sh.at[pl.ds(sid * DPT, DPT)],
                    out_hbm.at[cid, 0, pl.ds(sid * DPT, DPT)])


@functools.partial(
    pl.kernel,
    mesh=_mesh,
    out_type=jax.ShapeDtypeStruct((2, NPAD, DD), jnp.float32),
    scratch_types=[
        pltpu.VMEM((NBLK, BLK), jnp.int32),
        pltpu.VMEM((NBLK, BLK), jnp.int32),
        pltpu.VMEM((BLK, DD), jnp.float32),
        pltpu.VMEM_SHARED((NPAD, DD), jnp.float32),
        pltpu.SemaphoreType.DMA,
    ],
)
def _sc_edge(y_hbm, src_hbm, dst_hbm, z_hbm, out_hbm,
             src_v, dst_v, buf, acc_sh, sem):
    cid = lax.axis_index("c")
    sid = lax.axis_index("s")
    wid = cid * 16 + sid
    pltpu.sync_copy(src_hbm.at[wid], src_v)
    pltpu.sync_copy(dst_hbm.at[wid], dst_v)
    pltpu.sync_copy(z_hbm, acc_sh.at[pl.ds(sid * RPT, RPT)])
    plsc.subcore_barrier()

    def body(j, _):
        pltpu.async_copy(y_hbm.at[src_v.at[j]], buf, sem).wait()
        pltpu.sync_copy(buf, acc_sh.at[dst_v.at[j]], add=True)
        return 0

    lax.fori_loop(0, NBLK, body, 0)
    plsc.subcore_barrier()
    pltpu.sync_copy(acc_sh.at[pl.ds(sid * RPT, RPT)],
                    out_hbm.at[cid, pl.ds(sid * RPT, RPT)])


def _tc_deg_body(degp_ref, dinv_ref):
    deg = degp_ref[0] + degp_ref[1] + 1.0
    dinv_ref[...] = lax.rsqrt(deg)


def _tc_pre_body(x_ref, w_ref, dcol_ref, y_ref):
    y_ref[...] = jnp.dot(x_ref[...], w_ref[...],
                         preferred_element_type=jnp.float32) * dcol_ref[...]


def _elu(t):
    return jnp.where(t > 0, t, jnp.exp(jnp.minimum(t, 0.0)) - 1.0)


def _bn_elu(t, g_ref, be_ref):
    m = jnp.mean(t, axis=0, keepdims=True)
    v = jnp.mean((t - m) ** 2, axis=0, keepdims=True)
    hn = (t - m) * lax.rsqrt(v + 1e-5) * g_ref[...] + be_ref[...]
    return _elu(hn)


def _tc_post_body(accp_ref, y_ref, prev_ref, dcol_ref, b_ref, g_ref, be_ref,
                  wn_ref, h_ref, ynext_ref, *, has_prev):
    dcol = dcol_ref[...]
    t = dcol * (accp_ref[0, :NN] + accp_ref[1, :NN] + y_ref[...]) + b_ref[...]
    if has_prev:
        t = t + prev_ref[...]
    h = _bn_elu(t, g_ref, be_ref)
    h_ref[...] = h
    ynext_ref[...] = jnp.dot(h, wn_ref[...],
                             preferred_element_type=jnp.float32) * dcol


def _tc_final_body(accp_ref, y_ref, prev_ref, dcol_ref, b_ref, g_ref, be_ref,
                   batch_ref, wr_ref, br_ref, out_ref):
    dcol = dcol_ref[...]
    t = dcol * (accp_ref[0, :NN] + accp_ref[1, :NN] + y_ref[...]) + b_ref[...]
    t = t + prev_ref[...]
    h = _bn_elu(t, g_ref, be_ref)
    gidx = lax.broadcasted_iota(jnp.int32, (1, GG), 1)
    mask = (batch_ref[...] == gidx).astype(jnp.float32)
    counts = jnp.sum(mask, axis=0, keepdims=True)
    maskn = mask / jnp.maximum(counts, 1.0)
    pooled = lax.dot_general(maskn, h, (((0,), (0,)), ((), ())),
                             preferred_element_type=jnp.float32)
    out_ref[...] = jnp.dot(pooled, wr_ref[...],
                           preferred_element_type=jnp.float32) + br_ref[...]


_f32 = jnp.float32
_tc_deg = pl.pallas_call(
    _tc_deg_body, out_shape=jax.ShapeDtypeStruct((NPAD // DD, DD), _f32))
_tc_pre = pl.pallas_call(
    _tc_pre_body, out_shape=jax.ShapeDtypeStruct((NN, DD), _f32))
_tc_post0 = pl.pallas_call(
    functools.partial(_tc_post_body, has_prev=False),
    out_shape=(jax.ShapeDtypeStruct((NN, DD), _f32),
               jax.ShapeDtypeStruct((NN, DD), _f32)))
_tc_post1 = pl.pallas_call(
    functools.partial(_tc_post_body, has_prev=True),
    out_shape=(jax.ShapeDtypeStruct((NN, DD), _f32),
               jax.ShapeDtypeStruct((NN, DD), _f32)))
_tc_final = pl.pallas_call(
    _tc_final_body, out_shape=jax.ShapeDtypeStruct((GG, 2), _f32))


def kernel(x, edge_index, batch, W1, b1, g1, be1, W2, b2, g2, be2,
           W3, b3, g3, be3, Wr, br):
    src2d = edge_index[0].reshape(NW, NBLK, BLK)
    dst2d = edge_index[1].reshape(NW, NBLK, BLK)
    batch_col = batch.reshape(NN, 1)
    zrows = jnp.zeros((RPT, DD), _f32)
    b1r, g1r, be1r = b1.reshape(1, DD), g1.reshape(1, DD), be1.reshape(1, DD)
    b2r, g2r, be2r = b2.reshape(1, DD), g2.reshape(1, DD), be2.reshape(1, DD)
    b3r, g3r, be3r = b3.reshape(1, DD), g3.reshape(1, DD), be3.reshape(1, DD)
    brr = br.reshape(1, 2)

    degp = _sc_degree(dst2d)
    dinv_grid = _tc_deg(degp.reshape(2, NPAD // DD, DD))
    dcol = dinv_grid.reshape(NPAD, 1)[:NN]

    y1 = _tc_pre(x, W1, dcol)
    acc1 = _sc_edge(y1, src2d, dst2d, zrows)
    h1, y2 = _tc_post0(acc1, y1, y1, dcol, b1r, g1r, be1r, W2)
    acc2 = _sc_edge(y2, src2d, dst2d, zrows)
    h2, y3 = _tc_post1(acc2, y2, x, dcol, b2r, g2r, be2r, W3)
    acc3 = _sc_edge(y3, src2d, dst2d, zrows)
    out = _tc_final(acc3, y3, h1, dcol, b3r, g3r, be3r, batch_col, Wr, brr)
    return out


# speedup vs baseline: 19.1549x; 1.2228x over previous
import functools

import jax
import jax.numpy as jnp
from jax import lax
from jax.experimental import pallas as pl
from jax.experimental.pallas import tpu as pltpu
from jax.experimental.pallas import tpu_sc as plsc

NN = 10000
EE = 320000
DD = 128
GG = 128
NW = 32
EPW = EE // NW
BLK = 80
NBLK = EPW // BLK
NCH = 5
CHK = NBLK // NCH
NPAD = 10240
RPT = NPAD // 16
DPT = NPAD // 16

_mesh = plsc.VectorSubcoreMesh(core_axis_name="c", subcore_axis_name="s")


@functools.partial(
    pl.kernel,
    mesh=_mesh,
    out_type=jax.ShapeDtypeStruct((2, 1, NPAD), jnp.float32),
    scratch_types=[
        pltpu.VMEM((NCH, CHK, BLK), jnp.int32),
        pltpu.VMEM((BLK,), jnp.float32),
        pltpu.VMEM((DPT,), jnp.float32),
        pltpu.VMEM_SHARED((NPAD,), jnp.float32),
    ],
)
def _sc_degree(dst_hbm, out_hbm, dst_v, ones_v, zero_v, acc_sh):
    cid = lax.axis_index("c")
    sid = lax.axis_index("s")
    wid = cid * 16 + sid
    pltpu.sync_copy(dst_hbm.at[wid], dst_v)
    for i in range(BLK // 16):
        ones_v[pl.ds(i * 16, 16)] = jnp.full((16,), 1.0, jnp.float32)
    for i in range(DPT // 16):
        zero_v[pl.ds(i * 16, 16)] = jnp.zeros((16,), jnp.float32)
    pltpu.sync_copy(zero_v, acc_sh.at[pl.ds(sid * DPT, DPT)])
    plsc.subcore_barrier()

    def body(j, _):
        pltpu.sync_copy(ones_v, acc_sh.at[dst_v.at[j // CHK, j % CHK]],
                        add=True)
        return 0

    lax.fori_loop(0, NBLK, body, 0)
    plsc.subcore_barrier()
    pltpu.sync_copy(acc_sh.at[pl.ds(sid * DPT, DPT)],
                    out_hbm.at[cid, 0, pl.ds(sid * DPT, DPT)])


@functools.partial(
    pl.kernel,
    mesh=_mesh,
    out_type=jax.ShapeDtypeStruct((2, NPAD, DD), jnp.float32),
    scratch_types=[
        pltpu.VMEM((CHK, BLK), jnp.int32),
        pltpu.VMEM((CHK, BLK), jnp.int32),
        pltpu.VMEM((BLK, DD), jnp.float32),
        pltpu.VMEM((BLK, DD), jnp.float32),
        pltpu.VMEM_SHARED((NPAD, DD), jnp.float32),
        pltpu.SemaphoreType.DMA,
        pltpu.SemaphoreType.DMA,
    ],
)
def _sc_edge(y_hbm, src_hbm, dst_hbm, z_hbm, out_hbm,
             src_v, dst_v, buf0, buf1, acc_sh, sem0, sem1):
    cid = lax.axis_index("c")
    sid = lax.axis_index("s")
    wid = cid * 16 + sid
    pltpu.sync_copy(z_hbm, acc_sh.at[pl.ds(sid * RPT, RPT)])
    plsc.subcore_barrier()

    def chunk(c, _):
        pltpu.sync_copy(src_hbm.at[wid, c], src_v)
        pltpu.sync_copy(dst_hbm.at[wid, c], dst_v)
        pltpu.async_copy(y_hbm.at[src_v.at[0]], buf0, sem0)

        def body(k, _):
            j0 = 2 * k
            pltpu.make_async_copy(y_hbm.at[src_v.at[j0]], buf0, sem0).wait()
            pltpu.async_copy(y_hbm.at[src_v.at[j0 + 1]], buf1, sem1)
            pltpu.sync_copy(buf0, acc_sh.at[dst_v.at[j0]], add=True)
            pltpu.make_async_copy(
                y_hbm.at[src_v.at[j0 + 1]], buf1, sem1).wait()
            pltpu.async_copy(y_hbm.at[src_v.at[j0 + 2]], buf0, sem0)
            pltpu.sync_copy(buf1, acc_sh.at[dst_v.at[j0 + 1]], add=True)
            return 0

        lax.fori_loop(0, (CHK - 1) // 2, body, 0)
        pltpu.make_async_copy(y_hbm.at[src_v.at[CHK - 1]], buf0, sem0).wait()
        pltpu.sync_copy(buf0, acc_sh.at[dst_v.at[CHK - 1]], add=True)
        return 0

    lax.fori_loop(0, NCH, chunk, 0)
    plsc.subcore_barrier()
    pltpu.sync_copy(acc_sh.at[pl.ds(sid * RPT, RPT)],
                    out_hbm.at[cid, pl.ds(sid * RPT, RPT)])


def _tc_deg_body(degp_ref, dinv_ref):
    deg = degp_ref[0] + degp_ref[1] + 1.0
    dinv_ref[...] = lax.rsqrt(deg)


def _tc_pre_body(x_ref, w_ref, dcol_ref, y_ref):
    y_ref[...] = jnp.dot(x_ref[...], w_ref[...],
                         preferred_element_type=jnp.float32) * dcol_ref[...]


def _elu(t):
    return jnp.where(t > 0, t, jnp.exp(jnp.minimum(t, 0.0)) - 1.0)


def _bn_elu(t, g_ref, be_ref):
    m = jnp.mean(t, axis=0, keepdims=True)
    v = jnp.mean((t - m) ** 2, axis=0, keepdims=True)
    hn = (t - m) * lax.rsqrt(v + 1e-5) * g_ref[...] + be_ref[...]
    return _elu(hn)


def _tc_post_body(accp_ref, y_ref, prev_ref, dcol_ref, b_ref, g_ref, be_ref,
                  wn_ref, h_ref, ynext_ref, *, has_prev):
    dcol = dcol_ref[...]
    t = dcol * (accp_ref[0, :NN] + accp_ref[1, :NN] + y_ref[...]) + b_ref[...]
    if has_prev:
        t = t + prev_ref[...]
    h = _bn_elu(t, g_ref, be_ref)
    h_ref[...] = h
    ynext_ref[...] = jnp.dot(h, wn_ref[...],
                             preferred_element_type=jnp.float32) * dcol


def _tc_final_body(accp_ref, y_ref, prev_ref, dcol_ref, b_ref, g_ref, be_ref,
                   batch_ref, wr_ref, br_ref, out_ref):
    dcol = dcol_ref[...]
    t = dcol * (accp_ref[0, :NN] + accp_ref[1, :NN] + y_ref[...]) + b_ref[...]
    t = t + prev_ref[...]
    h = _bn_elu(t, g_ref, be_ref)
    gidx = lax.broadcasted_iota(jnp.int32, (1, GG), 1)
    mask = (batch_ref[...] == gidx).astype(jnp.float32)
    counts = jnp.sum(mask, axis=0, keepdims=True)
    maskn = mask / jnp.maximum(counts, 1.0)
    pooled = lax.dot_general(maskn, h, (((0,), (0,)), ((), ())),
                             preferred_element_type=jnp.float32)
    out_ref[...] = jnp.dot(pooled, wr_ref[...],
                           preferred_element_type=jnp.float32) + br_ref[...]


_f32 = jnp.float32
_tc_deg = pl.pallas_call(
    _tc_deg_body, out_shape=jax.ShapeDtypeStruct((NPAD // DD, DD), _f32))
_tc_pre = pl.pallas_call(
    _tc_pre_body, out_shape=jax.ShapeDtypeStruct((NN, DD), _f32))
_tc_post0 = pl.pallas_call(
    functools.partial(_tc_post_body, has_prev=False),
    out_shape=(jax.ShapeDtypeStruct((NN, DD), _f32),
               jax.ShapeDtypeStruct((NN, DD), _f32)))
_tc_post1 = pl.pallas_call(
    functools.partial(_tc_post_body, has_prev=True),
    out_shape=(jax.ShapeDtypeStruct((NN, DD), _f32),
               jax.ShapeDtypeStruct((NN, DD), _f32)))
_tc_final = pl.pallas_call(
    _tc_final_body, out_shape=jax.ShapeDtypeStruct((GG, 2), _f32))


def kernel(x, edge_index, batch, W1, b1, g1, be1, W2, b2, g2, be2,
           W3, b3, g3, be3, Wr, br):
    src2d = edge_index[0].reshape(NW, NCH, CHK, BLK)
    dst2d = edge_index[1].reshape(NW, NCH, CHK, BLK)
    batch_col = batch.reshape(NN, 1)
    zrows = jnp.zeros((RPT, DD), _f32)
    b1r, g1r, be1r = b1.reshape(1, DD), g1.reshape(1, DD), be1.reshape(1, DD)
    b2r, g2r, be2r = b2.reshape(1, DD), g2.reshape(1, DD), be2.reshape(1, DD)
    b3r, g3r, be3r = b3.reshape(1, DD), g3.reshape(1, DD), be3.reshape(1, DD)
    brr = br.reshape(1, 2)

    degp = _sc_degree(dst2d)
    dinv_grid = _tc_deg(degp.reshape(2, NPAD // DD, DD))
    dcol = dinv_grid.reshape(NPAD, 1)[:NN]

    y1 = _tc_pre(x, W1, dcol)
    acc1 = _sc_edge(y1, src2d, dst2d, zrows)
    h1, y2 = _tc_post0(acc1, y1, y1, dcol, b1r, g1r, be1r, W2)
    acc2 = _sc_edge(y2, src2d, dst2d, zrows)
    h2, y3 = _tc_post1(acc2, y2, x, dcol, b2r, g2r, be2r, W3)
    acc3 = _sc_edge(y3, src2d, dst2d, zrows)
    out = _tc_final(acc3, y3, h1, dcol, b3r, g3r, be3r, batch_col, Wr, brr)
    return out


# speedup vs baseline: 19.2850x; 1.0068x over previous
import functools

import jax
import jax.numpy as jnp
from jax import lax
from jax.experimental import pallas as pl
from jax.experimental.pallas import tpu as pltpu
from jax.experimental.pallas import tpu_sc as plsc

NN = 10000
EE = 320000
DD = 128
GG = 128
NW = 32
EPW = EE // NW
BLK = 80
NBLK = EPW // BLK
NCH = 5
CHK = NBLK // NCH
NPAD = 10240
RPT = NPAD // 16
DPT = NPAD // 16

_mesh = plsc.VectorSubcoreMesh(core_axis_name="c", subcore_axis_name="s")


@functools.partial(
    pl.kernel,
    mesh=_mesh,
    out_type=jax.ShapeDtypeStruct((2, 1, NPAD), jnp.float32),
    scratch_types=[
        pltpu.VMEM((NCH, CHK, BLK), jnp.int32),
        pltpu.VMEM((BLK,), jnp.float32),
        pltpu.VMEM((DPT,), jnp.float32),
        pltpu.VMEM_SHARED((NPAD,), jnp.float32),
    ],
)
def _sc_degree(dst_hbm, out_hbm, dst_v, ones_v, zero_v, acc_sh):
    cid = lax.axis_index("c")
    sid = lax.axis_index("s")
    wid = cid * 16 + sid
    pltpu.sync_copy(dst_hbm.at[wid], dst_v)
    for i in range(BLK // 16):
        ones_v[pl.ds(i * 16, 16)] = jnp.full((16,), 1.0, jnp.float32)
    for i in range(DPT // 16):
        zero_v[pl.ds(i * 16, 16)] = jnp.zeros((16,), jnp.float32)
    pltpu.sync_copy(zero_v, acc_sh.at[pl.ds(sid * DPT, DPT)])
    plsc.subcore_barrier()

    def body(j, _):
        pltpu.sync_copy(ones_v, acc_sh.at[dst_v.at[j // CHK, j % CHK]],
                        add=True)
        return 0

    lax.fori_loop(0, NBLK, body, 0)
    plsc.subcore_barrier()
    pltpu.sync_copy(acc_sh.at[pl.ds(sid * DPT, DPT)],
                    out_hbm.at[cid, 0, pl.ds(sid * DPT, DPT)])


@functools.partial(
    pl.kernel,
    mesh=_mesh,
    out_type=jax.ShapeDtypeStruct((2, NPAD, DD), jnp.float32),
    scratch_types=[
        pltpu.VMEM((CHK, BLK), jnp.int32),
        pltpu.VMEM((CHK, BLK), jnp.int32),
        pltpu.VMEM((BLK, DD), jnp.float32),
        pltpu.VMEM((BLK, DD), jnp.float32),
        pltpu.VMEM_SHARED((NPAD, DD), jnp.float32),
        pltpu.SemaphoreType.DMA,
        pltpu.SemaphoreType.DMA,
        pltpu.SemaphoreType.DMA,
        pltpu.SemaphoreType.DMA,
    ],
)
def _sc_edge(y_hbm, src_hbm, dst_hbm, z_hbm, out_hbm,
             src_v, dst_v, buf0, buf1, acc_sh, gsem0, gsem1, ssem0, ssem1):
    cid = lax.axis_index("c")
    sid = lax.axis_index("s")
    wid = cid * 16 + sid
    pltpu.sync_copy(z_hbm, acc_sh.at[pl.ds(sid * RPT, RPT)])
    plsc.subcore_barrier()

    def gwait(j, buf, sem):
        pltpu.make_async_copy(y_hbm.at[src_v.at[j]], buf, sem).wait()

    def swait(j, buf, sem):
        pltpu.make_async_copy(buf, acc_sh.at[dst_v.at[j]], sem).wait()

    def chunk(c, _):
        pltpu.sync_copy(src_hbm.at[wid, c], src_v)
        pltpu.sync_copy(dst_hbm.at[wid, c], dst_v)
        pltpu.async_copy(y_hbm.at[src_v.at[0]], buf0, gsem0)
        pltpu.async_copy(y_hbm.at[src_v.at[1]], buf1, gsem1)

        def body(k, _):
            j0 = 2 * k
            gwait(j0, buf0, gsem0)
            pltpu.async_copy(buf0, acc_sh.at[dst_v.at[j0]], ssem0, add=True)
            gwait(j0 + 1, buf1, gsem1)
            pltpu.async_copy(buf1, acc_sh.at[dst_v.at[j0 + 1]], ssem1,
                             add=True)
            swait(j0, buf0, ssem0)
            pltpu.async_copy(y_hbm.at[src_v.at[j0 + 2]], buf0, gsem0)
            swait(j0 + 1, buf1, ssem1)
            pltpu.async_copy(y_hbm.at[src_v.at[j0 + 3]], buf1, gsem1)
            return 0

        lax.fori_loop(0, (CHK - 3) // 2, body, 0)
        j = CHK - 3
        gwait(j, buf0, gsem0)
        pltpu.async_copy(buf0, acc_sh.at[dst_v.at[j]], ssem0, add=True)
        gwait(j + 1, buf1, gsem1)
        pltpu.async_copy(buf1, acc_sh.at[dst_v.at[j + 1]], ssem1, add=True)
        swait(j, buf0, ssem0)
        pltpu.async_copy(y_hbm.at[src_v.at[j + 2]], buf0, gsem0)
        gwait(j + 2, buf0, gsem0)
        pltpu.async_copy(buf0, acc_sh.at[dst_v.at[j + 2]], ssem0, add=True)
        swait(j + 1, buf1, ssem1)
        swait(j + 2, buf0, ssem0)
        return 0

    lax.fori_loop(0, NCH, chunk, 0)
    plsc.subcore_barrier()
    pltpu.sync_copy(acc_sh.at[pl.ds(sid * RPT, RPT)],
                    out_hbm.at[cid, pl.ds(sid * RPT, RPT)])


def _tc_deg_body(degp_ref, dinv_ref):
    deg = degp_ref[0] + degp_ref[1] + 1.0
    dinv_ref[...] = lax.rsqrt(deg)


def _tc_pre_body(x_ref, w_ref, dcol_ref, y_ref):
    y_ref[...] = jnp.dot(x_ref[...], w_ref[...],
                         preferred_element_type=jnp.float32) * dcol_ref[...]


def _elu(t):
    return jnp.where(t > 0, t, jnp.exp(jnp.minimum(t, 0.0)) - 1.0)


def _bn_elu(t, g_ref, be_ref):
    m = jnp.mean(t, axis=0, keepdims=True)
    v = jnp.mean((t - m) ** 2, axis=0, keepdims=True)
    hn = (t - m) * lax.rsqrt(v + 1e-5) * g_ref[...] + be_ref[...]
    return _elu(hn)


def _tc_post_body(accp_ref, y_ref, prev_ref, dcol_ref, b_ref, g_ref, be_ref,
                  wn_ref, h_ref, ynext_ref, *, has_prev):
    dcol = dcol_ref[...]
    t = dcol * (accp_ref[0, :NN] + accp_ref[1, :NN] + y_ref[...]) + b_ref[...]
    if has_prev:
        t = t + prev_ref[...]
    h = _bn_elu(t, g_ref, be_ref)
    h_ref[...] = h
    ynext_ref[...] = jnp.dot(h, wn_ref[...],
                             preferred_element_type=jnp.float32) * dcol


def _tc_final_body(accp_ref, y_ref, prev_ref, dcol_ref, b_ref, g_ref, be_ref,
                   batch_ref, wr_ref, br_ref, out_ref):
    dcol = dcol_ref[...]
    t = dcol * (accp_ref[0, :NN] + accp_ref[1, :NN] + y_ref[...]) + b_ref[...]
    t = t + prev_ref[...]
    h = _bn_elu(t, g_ref, be_ref)
    gidx = lax.broadcasted_iota(jnp.int32, (1, GG), 1)
    mask = (batch_ref[...] == gidx).astype(jnp.float32)
    counts = jnp.sum(mask, axis=0, keepdims=True)
    maskn = mask / jnp.maximum(counts, 1.0)
    pooled = lax.dot_general(maskn, h, (((0,), (0,)), ((), ())),
                             preferred_element_type=jnp.float32)
    out_ref[...] = jnp.dot(pooled, wr_ref[...],
                           preferred_element_type=jnp.float32) + br_ref[...]


_f32 = jnp.float32
_tc_deg = pl.pallas_call(
    _tc_deg_body, out_shape=jax.ShapeDtypeStruct((NPAD // DD, DD), _f32))
_tc_pre = pl.pallas_call(
    _tc_pre_body, out_shape=jax.ShapeDtypeStruct((NN, DD), _f32))
_tc_post0 = pl.pallas_call(
    functools.partial(_tc_post_body, has_prev=False),
    out_shape=(jax.ShapeDtypeStruct((NN, DD), _f32),
               jax.ShapeDtypeStruct((NN, DD), _f32)))
_tc_post1 = pl.pallas_call(
    functools.partial(_tc_post_body, has_prev=True),
    out_shape=(jax.ShapeDtypeStruct((NN, DD), _f32),
               jax.ShapeDtypeStruct((NN, DD), _f32)))
_tc_final = pl.pallas_call(
    _tc_final_body, out_shape=jax.ShapeDtypeStruct((GG, 2), _f32))


def kernel(x, edge_index, batch, W1, b1, g1, be1, W2, b2, g2, be2,
           W3, b3, g3, be3, Wr, br):
    src2d = edge_index[0].reshape(NW, NCH, CHK, BLK)
    dst2d = edge_index[1].reshape(NW, NCH, CHK, BLK)
    batch_col = batch.reshape(NN, 1)
    zrows = jnp.zeros((RPT, DD), _f32)
    b1r, g1r, be1r = b1.reshape(1, DD), g1.reshape(1, DD), be1.reshape(1, DD)
    b2r, g2r, be2r = b2.reshape(1, DD), g2.reshape(1, DD), be2.reshape(1, DD)
    b3r, g3r, be3r = b3.reshape(1, DD), g3.reshape(1, DD), be3.reshape(1, DD)
    brr = br.reshape(1, 2)

    degp = _sc_degree(dst2d)
    dinv_grid = _tc_deg(degp.reshape(2, NPAD // DD, DD))
    dcol = dinv_grid.reshape(NPAD, 1)[:NN]

    y1 = _tc_pre(x, W1, dcol)
    acc1 = _sc_edge(y1, src2d, dst2d, zrows)
    h1, y2 = _tc_post0(acc1, y1, y1, dcol, b1r, g1r, be1r, W2)
    acc2 = _sc_edge(y2, src2d, dst2d, zrows)
    h2, y3 = _tc_post1(acc2, y2, x, dcol, b2r, g2r, be2r, W3)
    acc3 = _sc_edge(y3, src2d, dst2d, zrows)
    out = _tc_final(acc3, y3, h1, dcol, b3r, g3r, be3r, batch_col, Wr, brr)
    return out
